# Initial kernel scaffold; baseline (speedup 1.0000x reference)
import jax
import jax.numpy as jnp
from jax.experimental import pallas as pl


def kernel(x, edge_index, edge_attr, batch, W1, b1, W2, b2, p, lw1, lb1, lw2, lb2, lw3, lb3):
    raise NotImplementedError("write your pallas kernel here")


# speedup vs baseline: 3.9213x; 3.9213x over previous
import functools
import jax
import jax.numpy as jnp
from jax import lax
from jax.experimental import pallas as pl
from jax.experimental.pallas import tpu as pltpu
from jax.experimental.pallas import tpu_sc as plsc

N = 100000
E = 1600000
G = 64
N2 = 100352
CHUNK = 80
F16 = 16

_f32 = jnp.float32
_bf16 = jnp.bfloat16


def _edge_agg_body(r_hbm, c_hbm, w_hbm, table_hbm, out_hbm,
                   rbuf, cbuf, wbuf, rows, obuf, acc, sem):
    cid = lax.axis_index("c")
    sid = lax.axis_index("s")
    per_worker = E // 2 // 16
    stripe = N2 // 16
    piece = stripe // 16

    obuf[...] = jnp.zeros((piece, F16), _f32)
    for j in range(16):
        pltpu.sync_copy(obuf, acc.at[pl.ds(sid * stripe + j * piece, piece)])
    plsc.subcore_barrier()

    base = cid * (E // 2) + sid * per_worker

    def body(it, carry):
        off = base + it * CHUNK
        pltpu.sync_copy(r_hbm.at[pl.ds(off, CHUNK)], rbuf)
        pltpu.sync_copy(c_hbm.at[pl.ds(off, CHUNK)], cbuf)
        pltpu.sync_copy(w_hbm.at[pl.ds(off, CHUNK)], wbuf)
        pltpu.async_copy(table_hbm.at[rbuf], rows, sem).wait()
        for i in range(CHUNK):
            rows[i, :] = rows[i, :] * wbuf[i, :]
        pltpu.async_copy(rows, acc.at[cbuf], sem, add=True).wait()
        return carry

    lax.fori_loop(0, per_worker // CHUNK, body, 0)
    plsc.subcore_barrier()

    for j in range(16):
        src = pl.ds(sid * stripe + j * piece, piece)
        pltpu.sync_copy(acc.at[src], obuf)
        pltpu.sync_copy(obuf, out_hbm.at[cid, src])


def _edge_agg(r, c, w, table):
    mesh = plsc.VectorSubcoreMesh(core_axis_name="c", subcore_axis_name="s")
    kern = functools.partial(
        pl.kernel,
        mesh=mesh,
        compiler_params=pltpu.CompilerParams(use_tc_tiling_on_sc=False),
        out_type=jax.ShapeDtypeStruct((2, N2, F16), _f32),
        scratch_types=[
            pltpu.VMEM((CHUNK,), jnp.int32),
            pltpu.VMEM((CHUNK,), jnp.int32),
            pltpu.VMEM((CHUNK, F16), _f32),
            pltpu.VMEM((CHUNK, F16), _f32),
            pltpu.VMEM((N2 // 256, F16), _f32),
            pltpu.VMEM_SHARED((N2, F16), _f32),
            pltpu.SemaphoreType.DMA,
        ],
    )(_edge_agg_body)
    return kern(r, c, w, table)


def _prep_body(deg_ref, x_ref, dinv_ref, xs_ref):
    deg = deg_ref[0] + deg_ref[1] + 1.0
    dinv = lax.rsqrt(deg)
    dinv_ref[...] = dinv
    xs_ref[...] = x_ref[...] * dinv


def _layer1_body(a1_ref, xs_ref, dinv_ref, w1_ref, b1_ref, h_ref, hs_ref):
    pre = dinv_ref[...] * (a1_ref[0] + a1_ref[1] + xs_ref[...])
    h = jnp.maximum(jnp.dot(pre, w1_ref[...],
                            preferred_element_type=_f32) + b1_ref[...], 0.0)
    h_ref[...] = h
    for q in range(4):
        hs_ref[q] = h[:, q * F16:(q + 1) * F16] * dinv_ref[...]


def _layer2_body(a20_ref, a21_ref, a22_ref, a23_ref, hs_ref, dinv_ref,
                 w2_ref, b2_ref, p_ref, h2_ref, sc_ref):
    parts = []
    for q, aref in enumerate((a20_ref, a21_ref, a22_ref, a23_ref)):
        parts.append(dinv_ref[...] * (aref[0] + aref[1] + hs_ref[q]))
    pre = jnp.concatenate(parts, axis=1)
    h2 = jnp.maximum(jnp.dot(pre, w2_ref[...],
                             preferred_element_type=_f32) + b2_ref[...], 0.0)
    h2_ref[...] = h2
    p = p_ref[...]
    pinv = lax.rsqrt(jnp.sum(p * p))
    sc_ref[...] = jnp.tanh(jnp.sum(h2 * p, axis=1, keepdims=True) * pinv)


def _topk_body(sc_ref, oht_ref, mask_ref, k_ref, cnt_ref):
    oht = oht_ref[...]
    uvals = sc_ref[...]
    b = lax.bitcast_convert_type(uvals, jnp.int32)
    u = jnp.where(b < 0, ~b, b | jnp.int32(-2147483648))
    uhi = lax.shift_right_logical(u, 16).astype(_f32)
    ulo = (u & jnp.int32(0xFFFF)).astype(_f32)

    ones = jnp.ones(uvals.shape, _bf16)
    counts = lax.dot_general(ones, oht, (((1,), (1,)), ((), ())),
                             preferred_element_type=_f32)
    k = jnp.ceil(0.8 * counts)
    cnt_ref[...] = counts
    k_ref[...] = k

    true_all = jnp.ones(uvals.shape, jnp.bool_)

    def gatherT(vec, planes):
        out = 0.0
        rem = vec
        for s in planes:
            hi = jnp.floor(rem * (1.0 / (1 << s)))
            rem = rem - hi * float(1 << s)
            out = out + float(1 << s) * lax.dot_general(
                hi.astype(_bf16), oht, (((1,), (0,)), ((), ())),
                preferred_element_type=_f32)
        return out + lax.dot_general(rem.astype(_bf16), oht,
                                     (((1,), (0,)), ((), ())),
                                     preferred_element_type=_f32)

    def count(pred):
        return lax.dot_general(pred.astype(_bf16), oht,
                               (((1,), (1,)), ((), ())),
                               preferred_element_type=_f32)

    def search_max(uvals_, kneed, part, nbits, planes):
        def body(_, lohi):
            lo, hi = lohi
            mid = jnp.floor((lo + hi + 1.0) * 0.5)
            pred = (uvals_ >= gatherT(mid, planes)) & part
            ok = count(pred) >= kneed
            return (jnp.where(ok, mid, lo), jnp.where(ok, hi, mid - 1.0))
        lo = jnp.zeros_like(kneed)
        hi = jnp.full_like(kneed, float((1 << nbits) - 1))
        lo, _ = lax.fori_loop(0, nbits, body, (lo, hi))
        return lo

    t_hi = search_max(uhi, k, true_all, 16, (8,))
    t_hi_pn = gatherT(t_hi, (8,))
    part2 = uhi == t_hi_pn
    gt_hi = uhi > t_hi_pn
    cnt_hi_gt = count(gt_hi)
    t_lo = search_max(ulo, k - cnt_hi_gt, part2, 16, (8,))
    t_lo_pn = gatherT(t_lo, (8,))
    gt = gt_hi | (part2 & (ulo > t_lo_pn))
    eq = part2 & (ulo == t_lo_pn)
    m = k - count(gt)

    def search_min_idx(kneed):
        def body(_, lohi):
            lo, hi = lohi
            mid = jnp.floor((lo + hi) * 0.5)
            idxf = lax.broadcasted_iota(jnp.int32, uvals.shape, 1).astype(_f32)
            pred = eq & (idxf < gatherT(mid, (16, 8)))
            ok = count(pred) >= kneed
            return (jnp.where(ok, lo, mid + 1.0), jnp.where(ok, mid, hi))
        lo = jnp.zeros_like(kneed)
        hi = jnp.full_like(kneed, float(N2))
        lo, hi = lax.fori_loop(0, 17, body, (lo, hi))
        return hi

    r = search_min_idx(m)
    r_pn = gatherT(r, (16, 8))
    idxf = lax.broadcasted_iota(jnp.int32, uvals.shape, 1).astype(_f32)
    mask_ref[...] = (gt | (eq & (idxf < r_pn))).astype(_f32)


def _final_body(h_ref, h2_ref, sc_ref, mask_ref, oh_ref, cnt_ref, k_ref,
                lw1_ref, lb1_ref, lw2_ref, lb2_ref, lw3_ref, lb3_ref,
                out_ref, acc1, acc2):
    i = pl.program_id(0)

    @pl.when(i == 0)
    def _():
        acc1[...] = jnp.zeros((G, 64), _f32)
        acc2[...] = jnp.zeros((G, 64), _f32)

    oh = oh_ref[...]
    acc1[...] += lax.dot_general(oh, h_ref[...], (((0,), (0,)), ((), ())),
                                 preferred_element_type=_f32)
    weighted = h2_ref[...] * (sc_ref[...] * mask_ref[...])
    acc2[...] += lax.dot_general(oh, weighted, (((0,), (0,)), ((), ())),
                                 preferred_element_type=_f32)

    @pl.when(i == pl.num_programs(0) - 1)
    def _():
        x1 = acc1[...] / jnp.maximum(cnt_ref[...], 1.0)
        x2 = acc2[...] / jnp.maximum(k_ref[...], 1.0)
        xg = x1 + x2
        xg = jnp.maximum(jnp.dot(xg, lw1_ref[...],
                                 preferred_element_type=_f32) + lb1_ref[...], 0.0)
        xg = jnp.maximum(jnp.dot(xg, lw2_ref[...],
                                 preferred_element_type=_f32) + lb2_ref[...], 0.0)
        z = jnp.dot(xg, lw3_ref[...],
                    preferred_element_type=_f32) + lb3_ref[...]
        out_ref[...] = 1.0 / (1.0 + jnp.exp(-z))


def kernel(x, edge_index, edge_attr, batch, W1, b1, W2, b2, p,
           lw1, lb1, lw2, lb2, lw3, lb3):
    r = edge_index[0]
    c = edge_index[1]
    w = jnp.broadcast_to(edge_attr[:, None], (E, F16))

    x_pad = jnp.zeros((N2, F16), _f32).at[:N, :3].set(x)
    ones16 = jnp.ones((N2, F16), _f32)
    batch_pad = jnp.concatenate(
        [batch, jnp.full((N2 - N,), G, jnp.int32)])
    oh_f32 = (batch_pad[:, None] == jnp.arange(G, dtype=jnp.int32)[None, :]
              ).astype(_f32)
    oht_bf = (jnp.arange(G, dtype=jnp.int32)[:, None] == batch_pad[None, :]
              ).astype(_bf16)

    deg_out = _edge_agg(r, c, w, ones16)

    grid16 = N2 // 1568
    blk = 1568

    dinv_rep, xs = pl.pallas_call(
        _prep_body,
        grid=(grid16,),
        in_specs=[
            pl.BlockSpec((2, blk, F16), lambda i: (0, i, 0)),
            pl.BlockSpec((blk, F16), lambda i: (i, 0)),
        ],
        out_specs=[
            pl.BlockSpec((blk, F16), lambda i: (i, 0)),
            pl.BlockSpec((blk, F16), lambda i: (i, 0)),
        ],
        out_shape=[
            jax.ShapeDtypeStruct((N2, F16), _f32),
            jax.ShapeDtypeStruct((N2, F16), _f32),
        ],
    )(deg_out, x_pad)

    a1 = _edge_agg(r, c, w, xs)

    W1p = jnp.zeros((F16, 64), _f32).at[:3].set(W1)

    h, hs = pl.pallas_call(
        _layer1_body,
        grid=(grid16,),
        in_specs=[
            pl.BlockSpec((2, blk, F16), lambda i: (0, i, 0)),
            pl.BlockSpec((blk, F16), lambda i: (i, 0)),
            pl.BlockSpec((blk, F16), lambda i: (i, 0)),
            pl.BlockSpec((F16, 64), lambda i: (0, 0)),
            pl.BlockSpec((1, 64), lambda i: (0, 0)),
        ],
        out_specs=[
            pl.BlockSpec((blk, 64), lambda i: (i, 0)),
            pl.BlockSpec((4, blk, F16), lambda i: (0, i, 0)),
        ],
        out_shape=[
            jax.ShapeDtypeStruct((N2, 64), _f32),
            jax.ShapeDtypeStruct((4, N2, F16), _f32),
        ],
    )(a1, xs, dinv_rep, W1p, b1.reshape(1, 64))

    a2 = [_edge_agg(r, c, w, hs[q]) for q in range(4)]

    h2, score = pl.pallas_call(
        _layer2_body,
        grid=(grid16,),
        in_specs=[pl.BlockSpec((2, blk, F16), lambda i: (0, i, 0))] * 4 + [
            pl.BlockSpec((4, blk, F16), lambda i: (0, i, 0)),
            pl.BlockSpec((blk, F16), lambda i: (i, 0)),
            pl.BlockSpec((64, 64), lambda i: (0, 0)),
            pl.BlockSpec((1, 64), lambda i: (0, 0)),
            pl.BlockSpec((1, 64), lambda i: (0, 0)),
        ],
        out_specs=[
            pl.BlockSpec((blk, 64), lambda i: (i, 0)),
            pl.BlockSpec((blk, 1), lambda i: (i, 0)),
        ],
        out_shape=[
            jax.ShapeDtypeStruct((N2, 64), _f32),
            jax.ShapeDtypeStruct((N2, 1), _f32),
        ],
    )(*a2, hs, dinv_rep, W2, b2.reshape(1, 64), p.reshape(1, 64))

    mask, kvec, counts = pl.pallas_call(
        _topk_body,
        out_shape=[
            jax.ShapeDtypeStruct((1, N2), _f32),
            jax.ShapeDtypeStruct((1, G), _f32),
            jax.ShapeDtypeStruct((1, G), _f32),
        ],
    )(score.reshape(1, N2), oht_bf)

    out = pl.pallas_call(
        _final_body,
        grid=(grid16,),
        in_specs=[
            pl.BlockSpec((blk, 64), lambda i: (i, 0)),
            pl.BlockSpec((blk, 64), lambda i: (i, 0)),
            pl.BlockSpec((blk, 1), lambda i: (i, 0)),
            pl.BlockSpec((blk, 1), lambda i: (i, 0)),
            pl.BlockSpec((blk, 64), lambda i: (i, 0)),
            pl.BlockSpec((G, 1), lambda i: (0, 0)),
            pl.BlockSpec((G, 1), lambda i: (0, 0)),
            pl.BlockSpec((64, 64), lambda i: (0, 0)),
            pl.BlockSpec((1, 64), lambda i: (0, 0)),
            pl.BlockSpec((64, 32), lambda i: (0, 0)),
            pl.BlockSpec((1, 32), lambda i: (0, 0)),
            pl.BlockSpec((32, 1), lambda i: (0, 0)),
            pl.BlockSpec((1, 1), lambda i: (0, 0)),
        ],
        out_specs=pl.BlockSpec((G, 1), lambda i: (0, 0)),
        out_shape=jax.ShapeDtypeStruct((G, 1), _f32),
        scratch_shapes=[pltpu.VMEM((G, 64), _f32), pltpu.VMEM((G, 64), _f32)],
    )(h, h2, score, mask.reshape(N2, 1), oh_f32,
      counts.reshape(G, 1), kvec.reshape(G, 1),
      lw1, lb1.reshape(1, 64), lw2, lb2.reshape(1, 32),
      lw3, lb3.reshape(1, 1))

    return out[:, 0]


# speedup vs baseline: 6.5950x; 1.6818x over previous
import functools
import jax
import jax.numpy as jnp
from jax import lax
from jax.experimental import pallas as pl
from jax.experimental.pallas import tpu as pltpu
from jax.experimental.pallas import tpu_sc as plsc

N = 100000
E = 1600000
G = 64
N2 = 100352
CHUNK = 80
INNER = 5
F16 = 16

_f32 = jnp.float32
_bf16 = jnp.bfloat16


def _edge_agg_body(r_hbm, c_hbm, w_hbm, table_hbm, out_hbm,
                   rbuf, cbuf, wbuf, rows, obuf, acc, sem, sem0, sem1):
    sems = (sem0, sem1)
    cid = lax.axis_index("c")
    sid = lax.axis_index("s")
    per_worker = E // 2 // 16
    stripe = N2 // 16
    piece = stripe // 16

    obuf[...] = jnp.zeros((piece, F16), _f32)
    for j in range(16):
        pltpu.sync_copy(obuf, acc.at[pl.ds(sid * stripe + j * piece, piece)])
    plsc.subcore_barrier()

    base_row = (cid * (E // 2) + sid * per_worker) // CHUNK

    cbuf[...] = jnp.zeros((INNER, CHUNK), jnp.int32)
    for p in range(2):
        rows[p] = jnp.zeros((CHUNK, F16), _f32)
        pltpu.async_copy(rows.at[p], acc.at[cbuf.at[0]], sems[p], add=True)

    def body(s, carry):
        off = base_row + s * INNER
        pltpu.sync_copy(r_hbm.at[pl.ds(off, INNER)], rbuf)
        pltpu.sync_copy(c_hbm.at[pl.ds(off, INNER)], cbuf)
        pltpu.sync_copy(w_hbm.at[pl.ds(off, INNER)], wbuf)
        for j in range(INNER):
            p = j % 2
            pltpu.make_async_copy(rows.at[p], acc.at[cbuf.at[j]],
                                  sems[p]).wait()
            pltpu.async_copy(table_hbm.at[rbuf.at[j]], rows.at[p],
                             sem).wait()
            for i in range(CHUNK):
                rows[p, i, :] = rows[p, i, :] * wbuf[j, i, :]
            pltpu.async_copy(rows.at[p], acc.at[cbuf.at[j]], sems[p],
                             add=True)
        return carry

    lax.fori_loop(0, per_worker // (CHUNK * INNER), body, 0)
    for p in range(2):
        pltpu.make_async_copy(rows.at[p], acc.at[cbuf.at[0]], sems[p]).wait()
    plsc.subcore_barrier()

    for j in range(16):
        src = pl.ds(sid * stripe + j * piece, piece)
        pltpu.sync_copy(acc.at[src], obuf)
        pltpu.sync_copy(obuf, out_hbm.at[cid, src])


def _edge_agg(r, c, w, table):
    mesh = plsc.VectorSubcoreMesh(core_axis_name="c", subcore_axis_name="s")
    kern = functools.partial(
        pl.kernel,
        mesh=mesh,
        compiler_params=pltpu.CompilerParams(use_tc_tiling_on_sc=False),
        out_type=jax.ShapeDtypeStruct((2, N2, F16), _f32),
        scratch_types=[
            pltpu.VMEM((INNER, CHUNK), jnp.int32),
            pltpu.VMEM((INNER, CHUNK), jnp.int32),
            pltpu.VMEM((INNER, CHUNK, F16), _f32),
            pltpu.VMEM((2, CHUNK, F16), _f32),
            pltpu.VMEM((N2 // 256, F16), _f32),
            pltpu.VMEM_SHARED((N2, F16), _f32),
            pltpu.SemaphoreType.DMA,
            pltpu.SemaphoreType.DMA,
            pltpu.SemaphoreType.DMA,
        ],
    )(_edge_agg_body)
    return kern(r, c, w, table)


def _prep_body(deg_ref, x_ref, dinv_ref, xs_ref):
    deg = deg_ref[0] + deg_ref[1] + 1.0
    dinv = lax.rsqrt(deg)
    dinv_ref[...] = dinv
    xs_ref[...] = x_ref[...] * dinv


def _layer1_body(a1_ref, xs_ref, dinv_ref, w1_ref, b1_ref, h_ref, hs_ref):
    pre = dinv_ref[...] * (a1_ref[0] + a1_ref[1] + xs_ref[...])
    h = jnp.maximum(jnp.dot(pre, w1_ref[...],
                            preferred_element_type=_f32) + b1_ref[...], 0.0)
    h_ref[...] = h
    for q in range(4):
        hs_ref[q] = h[:, q * F16:(q + 1) * F16] * dinv_ref[...]


def _layer2_body(a20_ref, a21_ref, a22_ref, a23_ref, hs_ref, dinv_ref,
                 w2_ref, b2_ref, p_ref, h2_ref, sc_ref):
    parts = []
    for q, aref in enumerate((a20_ref, a21_ref, a22_ref, a23_ref)):
        parts.append(dinv_ref[...] * (aref[0] + aref[1] + hs_ref[q]))
    pre = jnp.concatenate(parts, axis=1)
    h2 = jnp.maximum(jnp.dot(pre, w2_ref[...],
                             preferred_element_type=_f32) + b2_ref[...], 0.0)
    h2_ref[...] = h2
    p = p_ref[...]
    pinv = lax.rsqrt(jnp.sum(p * p))
    sc_ref[...] = jnp.tanh(jnp.sum(h2 * p, axis=1, keepdims=True) * pinv)


def _topk_body(sc_ref, oht_ref, mask_ref, k_ref, cnt_ref):
    oht = oht_ref[...]
    uvals = sc_ref[...]
    b = lax.bitcast_convert_type(uvals, jnp.int32)
    u = jnp.where(b < 0, ~b, b | jnp.int32(-2147483648))
    uhi = lax.shift_right_logical(u, 16).astype(_f32)
    ulo = (u & jnp.int32(0xFFFF)).astype(_f32)

    ones = jnp.ones(uvals.shape, _bf16)
    counts = lax.dot_general(ones, oht, (((1,), (1,)), ((), ())),
                             preferred_element_type=_f32)
    k = jnp.ceil(0.8 * counts)
    cnt_ref[...] = counts
    k_ref[...] = k

    true_all = jnp.ones(uvals.shape, jnp.bool_)

    def gatherT(vec, planes):
        out = 0.0
        rem = vec
        for s in planes:
            hi = jnp.floor(rem * (1.0 / (1 << s)))
            rem = rem - hi * float(1 << s)
            out = out + float(1 << s) * lax.dot_general(
                hi.astype(_bf16), oht, (((1,), (0,)), ((), ())),
                preferred_element_type=_f32)
        return out + lax.dot_general(rem.astype(_bf16), oht,
                                     (((1,), (0,)), ((), ())),
                                     preferred_element_type=_f32)

    def count(pred):
        return lax.dot_general(pred.astype(_bf16), oht,
                               (((1,), (1,)), ((), ())),
                               preferred_element_type=_f32)

    def search_max(uvals_, kneed, part, nbits, planes):
        def body(_, lohi):
            lo, hi = lohi
            mid = jnp.floor((lo + hi + 1.0) * 0.5)
            pred = (uvals_ >= gatherT(mid, planes)) & part
            ok = count(pred) >= kneed
            return (jnp.where(ok, mid, lo), jnp.where(ok, hi, mid - 1.0))
        lo = jnp.zeros_like(kneed)
        hi = jnp.full_like(kneed, float((1 << nbits) - 1))
        lo, _ = lax.fori_loop(0, nbits, body, (lo, hi))
        return lo

    t_hi = search_max(uhi, k, true_all, 16, (8,))
    t_hi_pn = gatherT(t_hi, (8,))
    part2 = uhi == t_hi_pn
    gt_hi = uhi > t_hi_pn
    cnt_hi_gt = count(gt_hi)
    t_lo = search_max(ulo, k - cnt_hi_gt, part2, 16, (8,))
    t_lo_pn = gatherT(t_lo, (8,))
    gt = gt_hi | (part2 & (ulo > t_lo_pn))
    eq = part2 & (ulo == t_lo_pn)
    m = k - count(gt)

    def search_min_idx(kneed):
        def body(_, lohi):
            lo, hi = lohi
            mid = jnp.floor((lo + hi) * 0.5)
            idxf = lax.broadcasted_iota(jnp.int32, uvals.shape, 1).astype(_f32)
            pred = eq & (idxf < gatherT(mid, (16, 8)))
            ok = count(pred) >= kneed
            return (jnp.where(ok, lo, mid + 1.0), jnp.where(ok, mid, hi))
        lo = jnp.zeros_like(kneed)
        hi = jnp.full_like(kneed, float(N2))
        lo, hi = lax.fori_loop(0, 17, body, (lo, hi))
        return hi

    r = search_min_idx(m)
    r_pn = gatherT(r, (16, 8))
    idxf = lax.broadcasted_iota(jnp.int32, uvals.shape, 1).astype(_f32)
    mask_ref[...] = (gt | (eq & (idxf < r_pn))).astype(_f32)


def _final_body(h_ref, h2_ref, sc_ref, mask_ref, oh_ref, cnt_ref, k_ref,
                lw1_ref, lb1_ref, lw2_ref, lb2_ref, lw3_ref, lb3_ref,
                out_ref, acc1, acc2):
    i = pl.program_id(0)

    @pl.when(i == 0)
    def _():
        acc1[...] = jnp.zeros((G, 64), _f32)
        acc2[...] = jnp.zeros((G, 64), _f32)

    oh = oh_ref[...]
    acc1[...] += lax.dot_general(oh, h_ref[...], (((0,), (0,)), ((), ())),
                                 preferred_element_type=_f32)
    weighted = h2_ref[...] * (sc_ref[...] * mask_ref[...])
    acc2[...] += lax.dot_general(oh, weighted, (((0,), (0,)), ((), ())),
                                 preferred_element_type=_f32)

    @pl.when(i == pl.num_programs(0) - 1)
    def _():
        x1 = acc1[...] / jnp.maximum(cnt_ref[...], 1.0)
        x2 = acc2[...] / jnp.maximum(k_ref[...], 1.0)
        xg = x1 + x2
        xg = jnp.maximum(jnp.dot(xg, lw1_ref[...],
                                 preferred_element_type=_f32) + lb1_ref[...], 0.0)
        xg = jnp.maximum(jnp.dot(xg, lw2_ref[...],
                                 preferred_element_type=_f32) + lb2_ref[...], 0.0)
        z = jnp.dot(xg, lw3_ref[...],
                    preferred_element_type=_f32) + lb3_ref[...]
        out_ref[...] = 1.0 / (1.0 + jnp.exp(-z))


def kernel(x, edge_index, edge_attr, batch, W1, b1, W2, b2, p,
           lw1, lb1, lw2, lb2, lw3, lb3):
    r = edge_index[0].reshape(E // CHUNK, CHUNK)
    c = edge_index[1].reshape(E // CHUNK, CHUNK)
    w = jnp.broadcast_to(edge_attr[:, None], (E, F16)).reshape(
        E // CHUNK, CHUNK, F16)

    x_pad = jnp.zeros((N2, F16), _f32).at[:N, :3].set(x)
    ones16 = jnp.ones((N2, F16), _f32)
    batch_pad = jnp.concatenate(
        [batch, jnp.full((N2 - N,), G, jnp.int32)])
    oh_f32 = (batch_pad[:, None] == jnp.arange(G, dtype=jnp.int32)[None, :]
              ).astype(_f32)
    oht_bf = (jnp.arange(G, dtype=jnp.int32)[:, None] == batch_pad[None, :]
              ).astype(_bf16)

    deg_out = _edge_agg(r, c, w, ones16)

    grid16 = N2 // 1568
    blk = 1568

    dinv_rep, xs = pl.pallas_call(
        _prep_body,
        grid=(grid16,),
        in_specs=[
            pl.BlockSpec((2, blk, F16), lambda i: (0, i, 0)),
            pl.BlockSpec((blk, F16), lambda i: (i, 0)),
        ],
        out_specs=[
            pl.BlockSpec((blk, F16), lambda i: (i, 0)),
            pl.BlockSpec((blk, F16), lambda i: (i, 0)),
        ],
        out_shape=[
            jax.ShapeDtypeStruct((N2, F16), _f32),
            jax.ShapeDtypeStruct((N2, F16), _f32),
        ],
    )(deg_out, x_pad)

    a1 = _edge_agg(r, c, w, xs)

    W1p = jnp.zeros((F16, 64), _f32).at[:3].set(W1)

    h, hs = pl.pallas_call(
        _layer1_body,
        grid=(grid16,),
        in_specs=[
            pl.BlockSpec((2, blk, F16), lambda i: (0, i, 0)),
            pl.BlockSpec((blk, F16), lambda i: (i, 0)),
            pl.BlockSpec((blk, F16), lambda i: (i, 0)),
            pl.BlockSpec((F16, 64), lambda i: (0, 0)),
            pl.BlockSpec((1, 64), lambda i: (0, 0)),
        ],
        out_specs=[
            pl.BlockSpec((blk, 64), lambda i: (i, 0)),
            pl.BlockSpec((4, blk, F16), lambda i: (0, i, 0)),
        ],
        out_shape=[
            jax.ShapeDtypeStruct((N2, 64), _f32),
            jax.ShapeDtypeStruct((4, N2, F16), _f32),
        ],
    )(a1, xs, dinv_rep, W1p, b1.reshape(1, 64))

    a2 = [_edge_agg(r, c, w, hs[q]) for q in range(4)]

    h2, score = pl.pallas_call(
        _layer2_body,
        grid=(grid16,),
        in_specs=[pl.BlockSpec((2, blk, F16), lambda i: (0, i, 0))] * 4 + [
            pl.BlockSpec((4, blk, F16), lambda i: (0, i, 0)),
            pl.BlockSpec((blk, F16), lambda i: (i, 0)),
            pl.BlockSpec((64, 64), lambda i: (0, 0)),
            pl.BlockSpec((1, 64), lambda i: (0, 0)),
            pl.BlockSpec((1, 64), lambda i: (0, 0)),
        ],
        out_specs=[
            pl.BlockSpec((blk, 64), lambda i: (i, 0)),
            pl.BlockSpec((blk, 1), lambda i: (i, 0)),
        ],
        out_shape=[
            jax.ShapeDtypeStruct((N2, 64), _f32),
            jax.ShapeDtypeStruct((N2, 1), _f32),
        ],
    )(*a2, hs, dinv_rep, W2, b2.reshape(1, 64), p.reshape(1, 64))

    mask, kvec, counts = pl.pallas_call(
        _topk_body,
        out_shape=[
            jax.ShapeDtypeStruct((1, N2), _f32),
            jax.ShapeDtypeStruct((1, G), _f32),
            jax.ShapeDtypeStruct((1, G), _f32),
        ],
    )(score.reshape(1, N2), oht_bf)

    out = pl.pallas_call(
        _final_body,
        grid=(grid16,),
        in_specs=[
            pl.BlockSpec((blk, 64), lambda i: (i, 0)),
            pl.BlockSpec((blk, 64), lambda i: (i, 0)),
            pl.BlockSpec((blk, 1), lambda i: (i, 0)),
            pl.BlockSpec((blk, 1), lambda i: (i, 0)),
            pl.BlockSpec((blk, 64), lambda i: (i, 0)),
            pl.BlockSpec((G, 1), lambda i: (0, 0)),
            pl.BlockSpec((G, 1), lambda i: (0, 0)),
            pl.BlockSpec((64, 64), lambda i: (0, 0)),
            pl.BlockSpec((1, 64), lambda i: (0, 0)),
            pl.BlockSpec((64, 32), lambda i: (0, 0)),
            pl.BlockSpec((1, 32), lambda i: (0, 0)),
            pl.BlockSpec((32, 1), lambda i: (0, 0)),
            pl.BlockSpec((1, 1), lambda i: (0, 0)),
        ],
        out_specs=pl.BlockSpec((G, 1), lambda i: (0, 0)),
        out_shape=jax.ShapeDtypeStruct((G, 1), _f32),
        scratch_shapes=[pltpu.VMEM((G, 64), _f32), pltpu.VMEM((G, 64), _f32)],
    )(h, h2, score, mask.reshape(N2, 1), oh_f32,
      counts.reshape(G, 1), kvec.reshape(G, 1),
      lw1, lb1.reshape(1, 64), lw2, lb2.reshape(1, 32),
      lw3, lb3.reshape(1, 1))

    return out[:, 0]


# speedup vs baseline: 8.7973x; 1.3339x over previous
import functools
import jax
import jax.numpy as jnp
from jax import lax
from jax.experimental import pallas as pl
from jax.experimental.pallas import tpu as pltpu
from jax.experimental.pallas import tpu_sc as plsc

N = 100000
E = 1600000
G = 64
N2 = 100352
CHUNK = 80
INNER = 5
F16 = 16

_f32 = jnp.float32
_bf16 = jnp.bfloat16


def _edge_agg_body(r_hbm, c_hbm, w_hbm, table_hbm, out_hbm,
                   rbuf, cbuf, wbuf, rows, obuf, acc, sem, sem_s, sem1):
    cid = lax.axis_index("c")
    sid = lax.axis_index("s")
    per_worker = E // 2 // 16
    stripe = N2 // 16
    piece = stripe // 16

    obuf[...] = jnp.zeros((piece, F16), _f32)
    for j in range(16):
        pltpu.sync_copy(obuf, acc.at[pl.ds(sid * stripe + j * piece, piece)])
    plsc.subcore_barrier()

    base_row = (cid * (E // 2) + sid * per_worker) // CHUNK

    cbuf[...] = jnp.zeros((INNER, CHUNK), jnp.int32)
    for j in range(INNER):
        rows[j] = jnp.zeros((CHUNK, F16), _f32)
        pltpu.async_copy(rows.at[j], acc.at[cbuf.at[j]], sem_s, add=True)

    def body(s, carry):
        off = base_row + s * INNER
        pltpu.sync_copy(r_hbm.at[pl.ds(off, INNER)], rbuf)
        for j in range(INNER):
            pltpu.make_async_copy(rows.at[j], acc.at[cbuf.at[j]],
                                  sem_s).wait()
        pltpu.sync_copy(c_hbm.at[pl.ds(off, INNER)], cbuf)
        pltpu.sync_copy(w_hbm.at[pl.ds(off, INNER)], wbuf)
        for j in range(INNER):
            pltpu.async_copy(table_hbm.at[rbuf.at[j]], rows.at[j], sem)
        for j in range(INNER):
            pltpu.make_async_copy(table_hbm.at[rbuf.at[j]], rows.at[j],
                                  sem).wait()
        for j in range(INNER):
            for i in range(CHUNK):
                rows[j, i, :] = rows[j, i, :] * wbuf[j, i, :]
            pltpu.async_copy(rows.at[j], acc.at[cbuf.at[j]], sem_s,
                             add=True)
        return carry

    lax.fori_loop(0, per_worker // (CHUNK * INNER), body, 0)
    for j in range(INNER):
        pltpu.make_async_copy(rows.at[j], acc.at[cbuf.at[j]], sem_s).wait()
    plsc.subcore_barrier()

    for j in range(16):
        src = pl.ds(sid * stripe + j * piece, piece)
        pltpu.sync_copy(acc.at[src], obuf)
        pltpu.sync_copy(obuf, out_hbm.at[cid, src])


def _edge_agg(r, c, w, table):
    mesh = plsc.VectorSubcoreMesh(core_axis_name="c", subcore_axis_name="s")
    kern = functools.partial(
        pl.kernel,
        mesh=mesh,
        compiler_params=pltpu.CompilerParams(use_tc_tiling_on_sc=False),
        out_type=jax.ShapeDtypeStruct((2, N2, F16), _f32),
        scratch_types=[
            pltpu.VMEM((INNER, CHUNK), jnp.int32),
            pltpu.VMEM((INNER, CHUNK), jnp.int32),
            pltpu.VMEM((INNER, CHUNK, F16), _f32),
            pltpu.VMEM((INNER, CHUNK, F16), _f32),
            pltpu.VMEM((N2 // 256, F16), _f32),
            pltpu.VMEM_SHARED((N2, F16), _f32),
            pltpu.SemaphoreType.DMA,
            pltpu.SemaphoreType.DMA,
            pltpu.SemaphoreType.DMA,
        ],
    )(_edge_agg_body)
    return kern(r, c, w, table)


def _prep_body(deg_ref, x_ref, dinv_ref, xs_ref):
    deg = deg_ref[0] + deg_ref[1] + 1.0
    dinv = lax.rsqrt(deg)
    dinv_ref[...] = dinv
    xs_ref[...] = x_ref[...] * dinv


def _layer1_body(a1_ref, xs_ref, dinv_ref, w1_ref, b1_ref, h_ref, hs_ref):
    pre = dinv_ref[...] * (a1_ref[0] + a1_ref[1] + xs_ref[...])
    h = jnp.maximum(jnp.dot(pre, w1_ref[...],
                            preferred_element_type=_f32) + b1_ref[...], 0.0)
    h_ref[...] = h
    for q in range(4):
        hs_ref[q] = h[:, q * F16:(q + 1) * F16] * dinv_ref[...]


def _layer2_body(a20_ref, a21_ref, a22_ref, a23_ref, hs_ref, dinv_ref,
                 w2_ref, b2_ref, p_ref, h2_ref, sc_ref):
    parts = []
    for q, aref in enumerate((a20_ref, a21_ref, a22_ref, a23_ref)):
        parts.append(dinv_ref[...] * (aref[0] + aref[1] + hs_ref[q]))
    pre = jnp.concatenate(parts, axis=1)
    h2 = jnp.maximum(jnp.dot(pre, w2_ref[...],
                             preferred_element_type=_f32) + b2_ref[...], 0.0)
    h2_ref[...] = h2
    p = p_ref[...]
    pinv = lax.rsqrt(jnp.sum(p * p))
    sc_ref[...] = jnp.tanh(jnp.sum(h2 * p, axis=1, keepdims=True) * pinv)


def _topk_body(sc_ref, oht_ref, mask_ref, k_ref, cnt_ref):
    oht = oht_ref[...]
    uvals = sc_ref[...]
    b = lax.bitcast_convert_type(uvals, jnp.int32)
    u = jnp.where(b < 0, ~b, b | jnp.int32(-2147483648))
    uhi = lax.shift_right_logical(u, 16).astype(_f32)
    ulo = (u & jnp.int32(0xFFFF)).astype(_f32)

    ones = jnp.ones(uvals.shape, _bf16)
    counts = lax.dot_general(ones, oht, (((1,), (1,)), ((), ())),
                             preferred_element_type=_f32)
    k = jnp.ceil(0.8 * counts)
    cnt_ref[...] = counts
    k_ref[...] = k

    true_all = jnp.ones(uvals.shape, jnp.bool_)

    def gatherT(vec, planes):
        out = 0.0
        rem = vec
        for s in planes:
            hi = jnp.floor(rem * (1.0 / (1 << s)))
            rem = rem - hi * float(1 << s)
            out = out + float(1 << s) * lax.dot_general(
                hi.astype(_bf16), oht, (((1,), (0,)), ((), ())),
                preferred_element_type=_f32)
        return out + lax.dot_general(rem.astype(_bf16), oht,
                                     (((1,), (0,)), ((), ())),
                                     preferred_element_type=_f32)

    def count(pred):
        return lax.dot_general(pred.astype(_bf16), oht,
                               (((1,), (1,)), ((), ())),
                               preferred_element_type=_f32)

    def search_max(uvals_, kneed, part, nbits, planes):
        def body(_, lohi):
            lo, hi = lohi
            mid = jnp.floor((lo + hi + 1.0) * 0.5)
            pred = (uvals_ >= gatherT(mid, planes)) & part
            ok = count(pred) >= kneed
            return (jnp.where(ok, mid, lo), jnp.where(ok, hi, mid - 1.0))
        lo = jnp.zeros_like(kneed)
        hi = jnp.full_like(kneed, float((1 << nbits) - 1))
        lo, _ = lax.fori_loop(0, nbits, body, (lo, hi))
        return lo

    t_hi = search_max(uhi, k, true_all, 16, (8,))
    t_hi_pn = gatherT(t_hi, (8,))
    part2 = uhi == t_hi_pn
    gt_hi = uhi > t_hi_pn
    cnt_hi_gt = count(gt_hi)
    t_lo = search_max(ulo, k - cnt_hi_gt, part2, 16, (8,))
    t_lo_pn = gatherT(t_lo, (8,))
    gt = gt_hi | (part2 & (ulo > t_lo_pn))
    eq = part2 & (ulo == t_lo_pn)
    m = k - count(gt)

    def search_min_idx(kneed):
        def body(_, lohi):
            lo, hi = lohi
            mid = jnp.floor((lo + hi) * 0.5)
            idxf = lax.broadcasted_iota(jnp.int32, uvals.shape, 1).astype(_f32)
            pred = eq & (idxf < gatherT(mid, (16, 8)))
            ok = count(pred) >= kneed
            return (jnp.where(ok, lo, mid + 1.0), jnp.where(ok, mid, hi))
        lo = jnp.zeros_like(kneed)
        hi = jnp.full_like(kneed, float(N2))
        lo, hi = lax.fori_loop(0, 17, body, (lo, hi))
        return hi

    r = search_min_idx(m)
    r_pn = gatherT(r, (16, 8))
    idxf = lax.broadcasted_iota(jnp.int32, uvals.shape, 1).astype(_f32)
    mask_ref[...] = (gt | (eq & (idxf < r_pn))).astype(_f32)


def _final_body(h_ref, h2_ref, sc_ref, mask_ref, oh_ref, cnt_ref, k_ref,
                lw1_ref, lb1_ref, lw2_ref, lb2_ref, lw3_ref, lb3_ref,
                out_ref, acc1, acc2):
    i = pl.program_id(0)

    @pl.when(i == 0)
    def _():
        acc1[...] = jnp.zeros((G, 64), _f32)
        acc2[...] = jnp.zeros((G, 64), _f32)

    oh = oh_ref[...]
    acc1[...] += lax.dot_general(oh, h_ref[...], (((0,), (0,)), ((), ())),
                                 preferred_element_type=_f32)
    weighted = h2_ref[...] * (sc_ref[...] * mask_ref[...])
    acc2[...] += lax.dot_general(oh, weighted, (((0,), (0,)), ((), ())),
                                 preferred_element_type=_f32)

    @pl.when(i == pl.num_programs(0) - 1)
    def _():
        x1 = acc1[...] / jnp.maximum(cnt_ref[...], 1.0)
        x2 = acc2[...] / jnp.maximum(k_ref[...], 1.0)
        xg = x1 + x2
        xg = jnp.maximum(jnp.dot(xg, lw1_ref[...],
                                 preferred_element_type=_f32) + lb1_ref[...], 0.0)
        xg = jnp.maximum(jnp.dot(xg, lw2_ref[...],
                                 preferred_element_type=_f32) + lb2_ref[...], 0.0)
        z = jnp.dot(xg, lw3_ref[...],
                    preferred_element_type=_f32) + lb3_ref[...]
        out_ref[...] = 1.0 / (1.0 + jnp.exp(-z))


def kernel(x, edge_index, edge_attr, batch, W1, b1, W2, b2, p,
           lw1, lb1, lw2, lb2, lw3, lb3):
    r = edge_index[0].reshape(E // CHUNK, CHUNK)
    c = edge_index[1].reshape(E // CHUNK, CHUNK)
    w = jnp.broadcast_to(edge_attr[:, None], (E, F16)).reshape(
        E // CHUNK, CHUNK, F16)

    x_pad = jnp.zeros((N2, F16), _f32).at[:N, :3].set(x)
    ones16 = jnp.ones((N2, F16), _f32)
    batch_pad = jnp.concatenate(
        [batch, jnp.full((N2 - N,), G, jnp.int32)])
    oh_f32 = (batch_pad[:, None] == jnp.arange(G, dtype=jnp.int32)[None, :]
              ).astype(_f32)
    oht_bf = (jnp.arange(G, dtype=jnp.int32)[:, None] == batch_pad[None, :]
              ).astype(_bf16)

    deg_out = _edge_agg(r, c, w, ones16)

    grid16 = N2 // 1568
    blk = 1568

    dinv_rep, xs = pl.pallas_call(
        _prep_body,
        grid=(grid16,),
        in_specs=[
            pl.BlockSpec((2, blk, F16), lambda i: (0, i, 0)),
            pl.BlockSpec((blk, F16), lambda i: (i, 0)),
        ],
        out_specs=[
            pl.BlockSpec((blk, F16), lambda i: (i, 0)),
            pl.BlockSpec((blk, F16), lambda i: (i, 0)),
        ],
        out_shape=[
            jax.ShapeDtypeStruct((N2, F16), _f32),
            jax.ShapeDtypeStruct((N2, F16), _f32),
        ],
    )(deg_out, x_pad)

    a1 = _edge_agg(r, c, w, xs)

    W1p = jnp.zeros((F16, 64), _f32).at[:3].set(W1)

    h, hs = pl.pallas_call(
        _layer1_body,
        grid=(grid16,),
        in_specs=[
            pl.BlockSpec((2, blk, F16), lambda i: (0, i, 0)),
            pl.BlockSpec((blk, F16), lambda i: (i, 0)),
            pl.BlockSpec((blk, F16), lambda i: (i, 0)),
            pl.BlockSpec((F16, 64), lambda i: (0, 0)),
            pl.BlockSpec((1, 64), lambda i: (0, 0)),
        ],
        out_specs=[
            pl.BlockSpec((blk, 64), lambda i: (i, 0)),
            pl.BlockSpec((4, blk, F16), lambda i: (0, i, 0)),
        ],
        out_shape=[
            jax.ShapeDtypeStruct((N2, 64), _f32),
            jax.ShapeDtypeStruct((4, N2, F16), _f32),
        ],
    )(a1, xs, dinv_rep, W1p, b1.reshape(1, 64))

    a2 = [_edge_agg(r, c, w, hs[q]) for q in range(4)]

    h2, score = pl.pallas_call(
        _layer2_body,
        grid=(grid16,),
        in_specs=[pl.BlockSpec((2, blk, F16), lambda i: (0, i, 0))] * 4 + [
            pl.BlockSpec((4, blk, F16), lambda i: (0, i, 0)),
            pl.BlockSpec((blk, F16), lambda i: (i, 0)),
            pl.BlockSpec((64, 64), lambda i: (0, 0)),
            pl.BlockSpec((1, 64), lambda i: (0, 0)),
            pl.BlockSpec((1, 64), lambda i: (0, 0)),
        ],
        out_specs=[
            pl.BlockSpec((blk, 64), lambda i: (i, 0)),
            pl.BlockSpec((blk, 1), lambda i: (i, 0)),
        ],
        out_shape=[
            jax.ShapeDtypeStruct((N2, 64), _f32),
            jax.ShapeDtypeStruct((N2, 1), _f32),
        ],
    )(*a2, hs, dinv_rep, W2, b2.reshape(1, 64), p.reshape(1, 64))

    mask, kvec, counts = pl.pallas_call(
        _topk_body,
        out_shape=[
            jax.ShapeDtypeStruct((1, N2), _f32),
            jax.ShapeDtypeStruct((1, G), _f32),
            jax.ShapeDtypeStruct((1, G), _f32),
        ],
    )(score.reshape(1, N2), oht_bf)

    out = pl.pallas_call(
        _final_body,
        grid=(grid16,),
        in_specs=[
            pl.BlockSpec((blk, 64), lambda i: (i, 0)),
            pl.BlockSpec((blk, 64), lambda i: (i, 0)),
            pl.BlockSpec((blk, 1), lambda i: (i, 0)),
            pl.BlockSpec((blk, 1), lambda i: (i, 0)),
            pl.BlockSpec((blk, 64), lambda i: (i, 0)),
            pl.BlockSpec((G, 1), lambda i: (0, 0)),
            pl.BlockSpec((G, 1), lambda i: (0, 0)),
            pl.BlockSpec((64, 64), lambda i: (0, 0)),
            pl.BlockSpec((1, 64), lambda i: (0, 0)),
            pl.BlockSpec((64, 32), lambda i: (0, 0)),
            pl.BlockSpec((1, 32), lambda i: (0, 0)),
            pl.BlockSpec((32, 1), lambda i: (0, 0)),
            pl.BlockSpec((1, 1), lambda i: (0, 0)),
        ],
        out_specs=pl.BlockSpec((G, 1), lambda i: (0, 0)),
        out_shape=jax.ShapeDtypeStruct((G, 1), _f32),
        scratch_shapes=[pltpu.VMEM((G, 64), _f32), pltpu.VMEM((G, 64), _f32)],
    )(h, h2, score, mask.reshape(N2, 1), oh_f32,
      counts.reshape(G, 1), kvec.reshape(G, 1),
      lw1, lb1.reshape(1, 64), lw2, lb2.reshape(1, 32),
      lw3, lb3.reshape(1, 1))

    return out[:, 0]
